# baseline (speedup 1.0000x reference)
import functools

import jax
import jax.numpy as jnp
from jax import lax
from jax.experimental import pallas as pl
from jax.experimental.pallas import tpu as pltpu
from jax.experimental.pallas import tpu_sc as plsc

N = 10000
E = 320000
D = 128
GH = 256
FH = 512
EPS = 1e-5

BLK = 1000
NB = N // BLK

SC_CORES = 2
SC_TILES = 16
CHUNK = 128
PH = 3
CPP = 54
CH = PH * CPP
EPT = CH * CHUNK
EPAD = SC_TILES * EPT
NPAD = 10240
ROWS_PER_TILE = NPAD // SC_TILES


def _dot_t(a, w):
    return lax.dot_general(a, w, (((1,), (1,)), ((), ())),
                           preferred_element_type=jnp.float32)


def _gelu(x):
    return 0.5 * x * (1.0 + lax.erf(x * 0.7071067811865476))


def _bn_from_psums(y, ps, pq, gamma, beta):
    mean = ps.sum(axis=(0, 1)) / N
    var = pq.sum(axis=(0, 1)) / N - mean * mean
    return gamma[0] * (y - mean) * lax.rsqrt(var + EPS) + beta[0]


def _psums(y, F):
    s = jnp.broadcast_to((y.sum(axis=0) / 8.0).reshape(1, 1, F), (1, 8, F))
    q = jnp.broadcast_to(((y * y).sum(axis=0) / 8.0).reshape(1, 1, F), (1, 8, F))
    return s, q


def _full(shape):
    nd = len(shape)
    return pl.BlockSpec(shape, lambda i: (0,) * nd)


def _blocked(F):
    return pl.BlockSpec((BLK, F), lambda i: (i, 0))


def _ps_spec(F):
    return pl.BlockSpec((1, 8, F), lambda i: (i, 0, 0))


def _tc1_body(x_ref, w_ref, b_ref, y_ref, ps_ref, pq_ref):
    y = _dot_t(x_ref[...], w_ref[...]) + b_ref[...]
    y_ref[...] = y
    ps_ref[...], pq_ref[...] = _psums(y, GH)


def _tc1(x, w, b):
    return pl.pallas_call(
        _tc1_body,
        grid=(NB,),
        in_specs=[_blocked(D), _full((GH, D)), _full((1, GH))],
        out_specs=[_blocked(GH), _ps_spec(GH), _ps_spec(GH)],
        out_shape=[
            jax.ShapeDtypeStruct((N, GH), jnp.float32),
            jax.ShapeDtypeStruct((NB, 8, GH), jnp.float32),
            jax.ShapeDtypeStruct((NB, 8, GH), jnp.float32),
        ],
    )(x, w, b)


def _tc2_body(y_ref, ps_ref, pq_ref, g_ref, b_ref, h_ref):
    h = _bn_from_psums(y_ref[...], ps_ref[...], pq_ref[...], g_ref[...], b_ref[...])
    h_ref[...] = jnp.stack([h[:, :D], h[:, D:]], axis=0)


def _tc2(y1, ps, pq, g, b):
    return pl.pallas_call(
        _tc2_body,
        grid=(NB,),
        in_specs=[_blocked(GH), _full((NB, 8, GH)), _full((NB, 8, GH)),
                  _full((1, GH)), _full((1, GH))],
        out_specs=[pl.BlockSpec((2, BLK, D), lambda i: (0, i, 0))],
        out_shape=[jax.ShapeDtypeStruct((2, N, D), jnp.float32)],
    )(y1, ps, pq, g, b)


def _sc_body(h_hbm, src_hbm, dst_hbm, out_hbm, acc, sem0, sem1):
    pl.run_scoped(
        functools.partial(_sc_inner, h_hbm, src_hbm, dst_hbm, out_hbm, acc,
                          sem0, sem1),
        pltpu.VMEM((CPP + 2, CHUNK), jnp.int32),
        pltpu.VMEM((CPP, CHUNK), jnp.int32),
        pltpu.VMEM((CHUNK, D), jnp.float32),
        pltpu.VMEM((CHUNK, D), jnp.float32),
    )


def _sc_inner(h_hbm, src_hbm, dst_hbm, out_hbm, acc, sem0, sem1,
              idx_v, dst_v, buf0, buf1):
    c = lax.axis_index("c")
    s = lax.axis_index("s")

    @pl.loop(0, CHUNK)
    def _zero(r):
        for g in range(D // 16):
            buf0[r, pl.ds(g * 16, 16)] = jnp.zeros((16,), jnp.float32)

    for k in range(ROWS_PER_TILE // CHUNK):
        pltpu.sync_copy(buf0, acc.at[pl.ds(s * ROWS_PER_TILE + k * CHUNK, CHUNK)])
    plsc.subcore_barrier()

    for p in range(PH):
        pltpu.sync_copy(src_hbm.at[c, s, p], idx_v)
        pltpu.sync_copy(dst_hbm.at[s, p], dst_v)

        pltpu.async_copy(h_hbm.at[idx_v.at[0]], buf0, sem0)
        pltpu.async_copy(h_hbm.at[idx_v.at[1]], buf1, sem1)

        @pl.loop(0, CPP, step=2)
        def _main(j):
            pltpu.make_async_copy(h_hbm.at[idx_v.at[j]], buf0, sem0).wait()
            pltpu.sync_copy(buf0, acc.at[dst_v.at[j]], add=True)
            pltpu.async_copy(h_hbm.at[idx_v.at[j + 2]], buf0, sem0)
            pltpu.make_async_copy(h_hbm.at[idx_v.at[j + 1]], buf1, sem1).wait()
            pltpu.sync_copy(buf1, acc.at[dst_v.at[j + 1]], add=True)
            pltpu.async_copy(h_hbm.at[idx_v.at[j + 3]], buf1, sem1)

        pltpu.make_async_copy(h_hbm.at[idx_v.at[CPP]], buf0, sem0).wait()
        pltpu.make_async_copy(h_hbm.at[idx_v.at[CPP + 1]], buf1, sem1).wait()

    plsc.subcore_barrier()
    pltpu.sync_copy(acc.at[pl.ds(s * ROWS_PER_TILE, ROWS_PER_TILE)],
                    out_hbm.at[c, pl.ds(s * ROWS_PER_TILE, ROWS_PER_TILE)])


@functools.cache
def _sc_agg_fn():
    return pl.kernel(
        _sc_body,
        out_type=jax.ShapeDtypeStruct((2, NPAD, D), jnp.float32),
        mesh=plsc.VectorSubcoreMesh(core_axis_name="c", subcore_axis_name="s",
                                    num_cores=SC_CORES, num_subcores=SC_TILES),
        scratch_types=[
            pltpu.VMEM_SHARED((NPAD, D), jnp.float32),
            pltpu.SemaphoreType.DMA,
            pltpu.SemaphoreType.DMA,
        ],
    )


def _tc3_body(agg_ref, h_ref, wrel_ref, brel_ref, wroot_ref, w2_ref, b2_ref,
              y_ref, ps_ref, pq_ref):
    gc = (_dot_t(agg_ref[0], wrel_ref[:, :D]) + _dot_t(agg_ref[1], wrel_ref[:, D:])
          + _dot_t(h_ref[0], wroot_ref[:, :D]) + _dot_t(h_ref[1], wroot_ref[:, D:])
          + brel_ref[...])
    y = _dot_t(_gelu(gc), w2_ref[...]) + b2_ref[...]
    y_ref[...] = y
    ps_ref[...], pq_ref[...] = _psums(y, D)


def _tc3(agg, h1s, wrel, brel, wroot, w2, b2):
    spec2 = pl.BlockSpec((2, BLK, D), lambda i: (0, i, 0))
    return pl.pallas_call(
        _tc3_body,
        grid=(NB,),
        in_specs=[spec2, spec2, _full((GH, GH)), _full((1, GH)),
                  _full((GH, GH)), _full((D, GH)), _full((1, D))],
        out_specs=[_blocked(D), _ps_spec(D), _ps_spec(D)],
        out_shape=[
            jax.ShapeDtypeStruct((N, D), jnp.float32),
            jax.ShapeDtypeStruct((NB, 8, D), jnp.float32),
            jax.ShapeDtypeStruct((NB, 8, D), jnp.float32),
        ],
    )(agg, h1s, wrel, brel, wroot, w2, b2)


def _tc4_body(y_ref, ps_ref, pq_ref, g_ref, b_ref, x_ref, w_ref, bf_ref,
              x1_ref, y3_ref, ps3_ref, pq3_ref):
    x1 = _bn_from_psums(y_ref[...], ps_ref[...], pq_ref[...],
                        g_ref[...], b_ref[...]) + x_ref[...]
    x1_ref[...] = x1
    y3 = _dot_t(x1, w_ref[...]) + bf_ref[...]
    y3_ref[...] = y3
    ps3_ref[...], pq3_ref[...] = _psums(y3, FH)


def _tc4(y2, ps, pq, g, b, x, w, bf):
    return pl.pallas_call(
        _tc4_body,
        grid=(NB,),
        in_specs=[_blocked(D), _full((NB, 8, D)), _full((NB, 8, D)),
                  _full((1, D)), _full((1, D)), _blocked(D),
                  _full((FH, D)), _full((1, FH))],
        out_specs=[_blocked(D), _blocked(FH), _ps_spec(FH), _ps_spec(FH)],
        out_shape=[
            jax.ShapeDtypeStruct((N, D), jnp.float32),
            jax.ShapeDtypeStruct((N, FH), jnp.float32),
            jax.ShapeDtypeStruct((NB, 8, FH), jnp.float32),
            jax.ShapeDtypeStruct((NB, 8, FH), jnp.float32),
        ],
    )(y2, ps, pq, g, b, x, w, bf)


def _tc5_body(y_ref, ps_ref, pq_ref, g_ref, b_ref, w_ref, bf_ref,
              y4_ref, ps4_ref, pq4_ref):
    h = _gelu(_bn_from_psums(y_ref[...], ps_ref[...], pq_ref[...],
                             g_ref[...], b_ref[...]))
    y4 = _dot_t(h, w_ref[...]) + bf_ref[...]
    y4_ref[...] = y4
    ps4_ref[...], pq4_ref[...] = _psums(y4, D)


def _tc5(y3, ps, pq, g, b, w, bf):
    return pl.pallas_call(
        _tc5_body,
        grid=(NB,),
        in_specs=[_blocked(FH), _full((NB, 8, FH)), _full((NB, 8, FH)),
                  _full((1, FH)), _full((1, FH)), _full((D, FH)), _full((1, D))],
        out_specs=[_blocked(D), _ps_spec(D), _ps_spec(D)],
        out_shape=[
            jax.ShapeDtypeStruct((N, D), jnp.float32),
            jax.ShapeDtypeStruct((NB, 8, D), jnp.float32),
            jax.ShapeDtypeStruct((NB, 8, D), jnp.float32),
        ],
    )(y3, ps, pq, g, b, w, bf)


def _tc6_body(y_ref, ps_ref, pq_ref, g_ref, b_ref, x1_ref, o_ref):
    o_ref[...] = _bn_from_psums(y_ref[...], ps_ref[...], pq_ref[...],
                                g_ref[...], b_ref[...]) + x1_ref[...]


def _tc6(y4, ps, pq, g, b, x1):
    return pl.pallas_call(
        _tc6_body,
        grid=(NB,),
        in_specs=[_blocked(D), _full((NB, 8, D)), _full((NB, 8, D)),
                  _full((1, D)), _full((1, D)), _blocked(D)],
        out_specs=[_blocked(D)],
        out_shape=[jax.ShapeDtypeStruct((N, D), jnp.float32)],
    )(y4, ps, pq, g, b, x1)


def kernel(x, edge_index, g_fc1_W, g_fc1_b, g_bn1_g, g_bn1_b, gc_Wrel, gc_brel,
           gc_Wroot, g_fc2_W, g_fc2_b, g_bn2_g, g_bn2_b, f_fc1_W, f_fc1_b,
           f_bn1_g, f_bn1_b, f_fc2_W, f_fc2_b, f_bn2_g, f_bn2_b):
    r1 = lambda v: v.reshape(1, -1)

    src = edge_index[0]
    dst = edge_index[1]
    order = jnp.argsort(dst)
    src = src[order]
    dst = dst[order]
    pad = EPAD - E
    srcp = jnp.concatenate([src, jnp.zeros((pad,), jnp.int32)])
    srcp = srcp.reshape(SC_TILES, PH, CPP, CHUNK)
    srcp = jnp.pad(srcp, ((0, 0), (0, 0), (0, 2), (0, 0)))
    src2 = jnp.stack([srcp, srcp + N])
    dstp = jnp.concatenate([dst, jnp.full((pad,), N, jnp.int32)])
    dstp = dstp.reshape(SC_TILES, PH, CPP, CHUNK)

    y1, ps1, pq1 = _tc1(x, g_fc1_W, r1(g_fc1_b))
    (h1s,) = _tc2(y1, ps1, pq1, r1(g_bn1_g), r1(g_bn1_b))
    aggp = _sc_agg_fn()(h1s.reshape(2 * N, D), src2, dstp)
    agg = aggp[:, :N, :]
    y2, ps2, pq2 = _tc3(agg, h1s, gc_Wrel, r1(gc_brel), gc_Wroot,
                        g_fc2_W, r1(g_fc2_b))
    x1, y3, ps3, pq3 = _tc4(y2, ps2, pq2, r1(g_bn2_g), r1(g_bn2_b), x,
                            f_fc1_W, r1(f_fc1_b))
    y4, ps4, pq4 = _tc5(y3, ps3, pq3, r1(f_bn1_g), r1(f_bn1_b),
                        f_fc2_W, r1(f_fc2_b))
    (out,) = _tc6(y4, ps4, pq4, r1(f_bn2_g), r1(f_bn2_b), x1)
    return out


# speedup vs baseline: 1.2015x; 1.2015x over previous
import functools

import jax
import jax.numpy as jnp
from jax import lax
from jax.experimental import pallas as pl
from jax.experimental.pallas import tpu as pltpu
from jax.experimental.pallas import tpu_sc as plsc

N = 10000
E = 320000
D = 128
GH = 256
FH = 512
EPS = 1e-5

BLK = 1000
NB = N // BLK

SC_CORES = 2
SC_TILES = 16
CHUNK = 64
SROW = 128
PH = 3
PR = 54
CH = PH * PR
EPT = CH * SROW
EPAD = SC_TILES * EPT
NPAD = 10240
ROWS_PER_TILE = NPAD // SC_TILES
NBLK = NPAD // 128


def _dot_t(a, w):
    return lax.dot_general(a, w, (((1,), (1,)), ((), ())),
                           preferred_element_type=jnp.float32)


def _gelu(x):
    return 0.5 * x * (1.0 + lax.erf(x * 0.7071067811865476))


def _bn_from_psums(y, ps, pq, gamma, beta):
    mean = ps.sum(axis=(0, 1)) / N
    var = pq.sum(axis=(0, 1)) / N - mean * mean
    return gamma[0] * (y - mean) * lax.rsqrt(var + EPS) + beta[0]


def _psums(y, F):
    s = jnp.broadcast_to((y.sum(axis=0) / 8.0).reshape(1, 1, F), (1, 8, F))
    q = jnp.broadcast_to(((y * y).sum(axis=0) / 8.0).reshape(1, 1, F), (1, 8, F))
    return s, q


def _full(shape):
    nd = len(shape)
    return pl.BlockSpec(shape, lambda i: (0,) * nd)


def _blocked(F):
    return pl.BlockSpec((BLK, F), lambda i: (i, 0))


def _ps_spec(F):
    return pl.BlockSpec((1, 8, F), lambda i: (i, 0, 0))


def _tc1_body(x_ref, w_ref, b_ref, y_ref, ps_ref, pq_ref):
    y = _dot_t(x_ref[...], w_ref[...]) + b_ref[...]
    y_ref[...] = y
    ps_ref[...], pq_ref[...] = _psums(y, GH)


def _tc1(x, w, b):
    return pl.pallas_call(
        _tc1_body,
        grid=(NB,),
        in_specs=[_blocked(D), _full((GH, D)), _full((1, GH))],
        out_specs=[_blocked(GH), _ps_spec(GH), _ps_spec(GH)],
        out_shape=[
            jax.ShapeDtypeStruct((N, GH), jnp.float32),
            jax.ShapeDtypeStruct((NB, 8, GH), jnp.float32),
            jax.ShapeDtypeStruct((NB, 8, GH), jnp.float32),
        ],
    )(x, w, b)


def _tc2_body(y_ref, ps_ref, pq_ref, g_ref, b_ref, h_ref):
    h = _bn_from_psums(y_ref[...], ps_ref[...], pq_ref[...], g_ref[...], b_ref[...])
    h_ref[...] = jnp.stack([h[:, :D], h[:, D:]], axis=0)


def _tc2(y1, ps, pq, g, b):
    return pl.pallas_call(
        _tc2_body,
        grid=(NB,),
        in_specs=[_blocked(GH), _full((NB, 8, GH)), _full((NB, 8, GH)),
                  _full((1, GH)), _full((1, GH))],
        out_specs=[pl.BlockSpec((2, BLK, D), lambda i: (0, i, 0))],
        out_shape=[jax.ShapeDtypeStruct((2, N, D), jnp.float32)],
    )(y1, ps, pq, g, b)


def _sc_body(h_hbm, src_hbm, dst_hbm, out_hbm, acc, sem0, sem1):
    pl.run_scoped(
        functools.partial(_sc_inner, h_hbm, src_hbm, dst_hbm, out_hbm, acc,
                          sem0, sem1),
        pltpu.VMEM((PR + 1, SROW), jnp.int32),
        pltpu.VMEM((PR, SROW), jnp.int32),
        pltpu.VMEM((CHUNK, D), jnp.float32),
        pltpu.VMEM((CHUNK, D), jnp.float32),
        pltpu.VMEM((128, D), jnp.float32),
        pltpu.VMEM((1, 128), jnp.int32),
    )


def _zero_staging(stg):
    @pl.loop(0, 128)
    def _zr(r):
        for g in range(D // 16):
            stg[r, pl.ds(g * 16, 16)] = jnp.zeros((16,), jnp.float32)


def _sc_inner(h_hbm, src_hbm, dst_hbm, out_hbm, acc, sem0, sem1,
              idx_v, dst_v, buf0, buf1, stg, fli):
    c = lax.axis_index("c")
    s = lax.axis_index("s")

    def flush(blk):
        base = blk * 128
        ramp = lax.iota(jnp.int32, 16)
        for g in range(8):
            fli[0, pl.ds(g * 16, 16)] = ramp + (base + g * 16)
        pltpu.sync_copy(stg, acc.at[fli.at[0]], add=True)
        _zero_staging(stg)

    _zero_staging(stg)
    for k in range(ROWS_PER_TILE // 128):
        pltpu.sync_copy(stg, acc.at[pl.ds(s * ROWS_PER_TILE + k * 128, 128)])
    plsc.subcore_barrier()

    zero16 = jnp.zeros((16,), jnp.float32)

    def consume(buf, jr, b, carry):
        @pl.loop(0, CHUNK // 16, init_carry=carry)
        def group_loop(q, st):
            dvec = dst_v[jr, pl.ds(b * CHUNK + q * 16, 16)]
            blkvec = lax.shift_right_arithmetic(dvec, 7)
            rlvec = lax.bitwise_and(dvec, 127)
            for i in range(16):
                acc8 = st[:8]
                d_prev = st[8]
                d = dvec[i]
                blk = blkvec[i]
                pblk = lax.shift_right_arithmetic(d_prev, 7)

                @pl.when(jnp.logical_and(d_prev >= 0, blk != pblk))
                def _():
                    flush(pblk)

                same = d == d_prev
                rloc = rlvec[i]
                new8 = []
                for g in range(8):
                    row = buf[q * 16 + i, pl.ds(g * 16, 16)]
                    a = jnp.where(same, acc8[g], zero16) + row
                    stg[rloc, pl.ds(g * 16, 16)] = a
                    new8.append(a)
                st = (*new8, d)
            return st

        return group_loop

    carry = tuple([zero16] * 8) + (jnp.int32(-1),)

    for p in range(PH):
        pltpu.sync_copy(src_hbm.at[c, s, p], idx_v)
        pltpu.sync_copy(dst_hbm.at[s, p], dst_v)

        pltpu.async_copy(h_hbm.at[idx_v.at[0, pl.ds(0, CHUNK)]], buf0, sem0)
        pltpu.async_copy(h_hbm.at[idx_v.at[0, pl.ds(CHUNK, CHUNK)]], buf1, sem1)

        @pl.loop(0, PR, init_carry=carry)
        def row_loop(jr, st):
            pltpu.make_async_copy(h_hbm.at[idx_v.at[0, pl.ds(0, CHUNK)]],
                                  buf0, sem0).wait()
            st = consume(buf0, jr, 0, st)
            pltpu.async_copy(h_hbm.at[idx_v.at[jr + 1, pl.ds(0, CHUNK)]],
                             buf0, sem0)
            pltpu.make_async_copy(h_hbm.at[idx_v.at[0, pl.ds(CHUNK, CHUNK)]],
                                  buf1, sem1).wait()
            st = consume(buf1, jr, 1, st)
            pltpu.async_copy(h_hbm.at[idx_v.at[jr + 1, pl.ds(CHUNK, CHUNK)]],
                             buf1, sem1)
            return st

        carry = row_loop
        pltpu.make_async_copy(h_hbm.at[idx_v.at[0, pl.ds(0, CHUNK)]],
                              buf0, sem0).wait()
        pltpu.make_async_copy(h_hbm.at[idx_v.at[0, pl.ds(CHUNK, CHUNK)]],
                              buf1, sem1).wait()

    flush(lax.shift_right_logical(carry[8], 7))

    plsc.subcore_barrier()
    pltpu.sync_copy(acc.at[pl.ds(s * ROWS_PER_TILE, ROWS_PER_TILE)],
                    out_hbm.at[c, pl.ds(s * ROWS_PER_TILE, ROWS_PER_TILE)])


@functools.cache
def _sc_agg_fn():
    return pl.kernel(
        _sc_body,
        out_type=jax.ShapeDtypeStruct((2, NPAD, D), jnp.float32),
        mesh=plsc.VectorSubcoreMesh(core_axis_name="c", subcore_axis_name="s",
                                    num_cores=SC_CORES, num_subcores=SC_TILES),
        scratch_types=[
            pltpu.VMEM_SHARED((NPAD, D), jnp.float32),
            pltpu.SemaphoreType.DMA,
            pltpu.SemaphoreType.DMA,
        ],
    )


def _tc3_body(agg_ref, h_ref, wrel_ref, brel_ref, wroot_ref, w2_ref, b2_ref,
              y_ref, ps_ref, pq_ref):
    gc = (_dot_t(agg_ref[0], wrel_ref[:, :D]) + _dot_t(agg_ref[1], wrel_ref[:, D:])
          + _dot_t(h_ref[0], wroot_ref[:, :D]) + _dot_t(h_ref[1], wroot_ref[:, D:])
          + brel_ref[...])
    y = _dot_t(_gelu(gc), w2_ref[...]) + b2_ref[...]
    y_ref[...] = y
    ps_ref[...], pq_ref[...] = _psums(y, D)


def _tc3(agg, h1s, wrel, brel, wroot, w2, b2):
    spec2 = pl.BlockSpec((2, BLK, D), lambda i: (0, i, 0))
    return pl.pallas_call(
        _tc3_body,
        grid=(NB,),
        in_specs=[spec2, spec2, _full((GH, GH)), _full((1, GH)),
                  _full((GH, GH)), _full((D, GH)), _full((1, D))],
        out_specs=[_blocked(D), _ps_spec(D), _ps_spec(D)],
        out_shape=[
            jax.ShapeDtypeStruct((N, D), jnp.float32),
            jax.ShapeDtypeStruct((NB, 8, D), jnp.float32),
            jax.ShapeDtypeStruct((NB, 8, D), jnp.float32),
        ],
    )(agg, h1s, wrel, brel, wroot, w2, b2)


def _tc4_body(y_ref, ps_ref, pq_ref, g_ref, b_ref, x_ref, w_ref, bf_ref,
              x1_ref, y3_ref, ps3_ref, pq3_ref):
    x1 = _bn_from_psums(y_ref[...], ps_ref[...], pq_ref[...],
                        g_ref[...], b_ref[...]) + x_ref[...]
    x1_ref[...] = x1
    y3 = _dot_t(x1, w_ref[...]) + bf_ref[...]
    y3_ref[...] = y3
    ps3_ref[...], pq3_ref[...] = _psums(y3, FH)


def _tc4(y2, ps, pq, g, b, x, w, bf):
    return pl.pallas_call(
        _tc4_body,
        grid=(NB,),
        in_specs=[_blocked(D), _full((NB, 8, D)), _full((NB, 8, D)),
                  _full((1, D)), _full((1, D)), _blocked(D),
                  _full((FH, D)), _full((1, FH))],
        out_specs=[_blocked(D), _blocked(FH), _ps_spec(FH), _ps_spec(FH)],
        out_shape=[
            jax.ShapeDtypeStruct((N, D), jnp.float32),
            jax.ShapeDtypeStruct((N, FH), jnp.float32),
            jax.ShapeDtypeStruct((NB, 8, FH), jnp.float32),
            jax.ShapeDtypeStruct((NB, 8, FH), jnp.float32),
        ],
    )(y2, ps, pq, g, b, x, w, bf)


def _tc5_body(y_ref, ps_ref, pq_ref, g_ref, b_ref, w_ref, bf_ref,
              y4_ref, ps4_ref, pq4_ref):
    h = _gelu(_bn_from_psums(y_ref[...], ps_ref[...], pq_ref[...],
                             g_ref[...], b_ref[...]))
    y4 = _dot_t(h, w_ref[...]) + bf_ref[...]
    y4_ref[...] = y4
    ps4_ref[...], pq4_ref[...] = _psums(y4, D)


def _tc5(y3, ps, pq, g, b, w, bf):
    return pl.pallas_call(
        _tc5_body,
        grid=(NB,),
        in_specs=[_blocked(FH), _full((NB, 8, FH)), _full((NB, 8, FH)),
                  _full((1, FH)), _full((1, FH)), _full((D, FH)), _full((1, D))],
        out_specs=[_blocked(D), _ps_spec(D), _ps_spec(D)],
        out_shape=[
            jax.ShapeDtypeStruct((N, D), jnp.float32),
            jax.ShapeDtypeStruct((NB, 8, D), jnp.float32),
            jax.ShapeDtypeStruct((NB, 8, D), jnp.float32),
        ],
    )(y3, ps, pq, g, b, w, bf)


def _tc6_body(y_ref, ps_ref, pq_ref, g_ref, b_ref, x1_ref, o_ref):
    o_ref[...] = _bn_from_psums(y_ref[...], ps_ref[...], pq_ref[...],
                                g_ref[...], b_ref[...]) + x1_ref[...]


def _tc6(y4, ps, pq, g, b, x1):
    return pl.pallas_call(
        _tc6_body,
        grid=(NB,),
        in_specs=[_blocked(D), _full((NB, 8, D)), _full((NB, 8, D)),
                  _full((1, D)), _full((1, D)), _blocked(D)],
        out_specs=[_blocked(D)],
        out_shape=[jax.ShapeDtypeStruct((N, D), jnp.float32)],
    )(y4, ps, pq, g, b, x1)


def kernel(x, edge_index, g_fc1_W, g_fc1_b, g_bn1_g, g_bn1_b, gc_Wrel, gc_brel,
           gc_Wroot, g_fc2_W, g_fc2_b, g_bn2_g, g_bn2_b, f_fc1_W, f_fc1_b,
           f_bn1_g, f_bn1_b, f_fc2_W, f_fc2_b, f_bn2_g, f_bn2_b):
    r1 = lambda v: v.reshape(1, -1)

    src = edge_index[0]
    dst = edge_index[1]
    order = jnp.argsort(dst)
    src = src[order]
    dst = dst[order]
    pad = EPAD - E
    srcp = jnp.concatenate([src, jnp.zeros((pad,), jnp.int32)])
    srcp = srcp.reshape(SC_TILES, CH, SROW)
    srcp = jnp.pad(srcp, ((0, 0), (0, 1), (0, 0)))
    srcp = jnp.stack([srcp[:, PR * p:PR * p + PR + 1] for p in range(PH)],
                     axis=1)
    src2 = jnp.stack([srcp, srcp + N])
    dstp = jnp.concatenate([dst, jnp.full((pad,), N, jnp.int32)])
    dstp = dstp.reshape(SC_TILES, PH, PR, SROW)

    y1, ps1, pq1 = _tc1(x, g_fc1_W, r1(g_fc1_b))
    (h1s,) = _tc2(y1, ps1, pq1, r1(g_bn1_g), r1(g_bn1_b))
    aggp = _sc_agg_fn()(h1s.reshape(2 * N, D), src2, dstp)
    agg = aggp[:, :N, :]
    y2, ps2, pq2 = _tc3(agg, h1s, gc_Wrel, r1(gc_brel), gc_Wroot,
                        g_fc2_W, r1(g_fc2_b))
    x1, y3, ps3, pq3 = _tc4(y2, ps2, pq2, r1(g_bn2_g), r1(g_bn2_b), x,
                            f_fc1_W, r1(f_fc1_b))
    y4, ps4, pq4 = _tc5(y3, ps3, pq3, r1(f_bn1_g), r1(f_bn1_b),
                        f_fc2_W, r1(f_fc2_b))
    (out,) = _tc6(y4, ps4, pq4, r1(f_bn2_g), r1(f_bn2_b), x1)
    return out
